# baseline (speedup 1.0000x reference)
import functools
import math

import jax
import jax.numpy as jnp
from jax import lax
from jax.experimental import pallas as pl
from jax.experimental.pallas import tpu as pltpu
from jax.experimental.pallas import tpu_sc as plsc

B0, B1 = 4096, 200
D = 64
NW = 32
B = B0 * B1
PER_W = B // NW
CHUNK = 128
GROUP = 2 * CHUNK
NG = PER_W // GROUP
SCALE = math.sqrt(D)

_mesh = plsc.VectorSubcoreMesh(core_axis_name="c", subcore_axis_name="s")


@functools.partial(
    pl.kernel,
    mesh=_mesh,
    compiler_params=pltpu.CompilerParams(use_tc_tiling_on_sc=False),
    out_type=jax.ShapeDtypeStruct((NW, NG, GROUP, D), jnp.float32),
    scratch_types=[
        pltpu.VMEM((NG, GROUP), jnp.int32),
        pltpu.VMEM((GROUP, D), jnp.float32),
        pltpu.SemaphoreType.DMA,
    ],
)
def _emb_lookup(x_hbm, tab_hbm, out_hbm, idx_v, buf, gsem):
    wid = lax.axis_index("s") * 2 + lax.axis_index("c")
    pltpu.sync_copy(x_hbm.at[wid], idx_v)

    def body(g, _):
        cps = []
        for j in range(GROUP // CHUNK):
            cps.append(
                pltpu.async_copy(
                    tab_hbm.at[idx_v.at[g, pl.ds(j * CHUNK, CHUNK)]],
                    buf.at[pl.ds(j * CHUNK, CHUNK)],
                    gsem,
                )
            )
        for cp in cps:
            cp.wait()

        def srow(r, _):
            for j in range(D // 16):
                sl = pl.ds(j * 16, 16)
                buf[r, sl] = buf[r, sl] * SCALE
            return ()

        lax.fori_loop(0, GROUP, srow, (), unroll=4)

        pltpu.sync_copy(buf, out_hbm.at[wid, g])
        return ()

    lax.fori_loop(0, NG, body, ())


def kernel(x, table):
    xs = x.reshape(NW, NG, GROUP).astype(jnp.int32)
    out = _emb_lookup(xs, table)
    return out.reshape(B0, B1, D)


# speedup vs baseline: 1.1005x; 1.1005x over previous
import functools
import math

import jax
import jax.numpy as jnp
from jax import lax
from jax.experimental import pallas as pl
from jax.experimental.pallas import tpu as pltpu
from jax.experimental.pallas import tpu_sc as plsc

B0, B1 = 4096, 200
D = 64
NW = 32
B = B0 * B1
PER_W = B // NW
CHUNK = 128
GROUP = 2 * CHUNK
NG = PER_W // GROUP
NBUF = 4
SCALE = math.sqrt(D)

_mesh = plsc.VectorSubcoreMesh(core_axis_name="c", subcore_axis_name="s")


@functools.partial(
    pl.kernel,
    mesh=_mesh,
    compiler_params=pltpu.CompilerParams(use_tc_tiling_on_sc=False),
    out_type=jax.ShapeDtypeStruct((NW, NG, GROUP, D), jnp.float32),
    scratch_types=[
        pltpu.VMEM((NG, GROUP), jnp.int32),
    ]
    + [pltpu.VMEM((GROUP, D), jnp.float32) for _ in range(NBUF)]
    + [pltpu.SemaphoreType.DMA for _ in range(2 * NBUF)],
)
def _emb_lookup(x_hbm, tab_hbm, out_hbm, idx_v, *bufs_and_sems):
    bufs = bufs_and_sems[:NBUF]
    gsem = bufs_and_sems[NBUF:2 * NBUF]
    ssem = bufs_and_sems[2 * NBUF:]
    wid = lax.axis_index("s") * 2 + lax.axis_index("c")
    pltpu.sync_copy(x_hbm.at[wid], idx_v)

    def fire_gather(g, i):
        for j in range(GROUP // CHUNK):
            pltpu.async_copy(
                tab_hbm.at[idx_v.at[g, pl.ds(j * CHUNK, CHUNK)]],
                bufs[i].at[pl.ds(j * CHUNK, CHUNK)],
                gsem[i],
            )

    def drain_gather(i):
        for j in range(GROUP // CHUNK):
            pltpu.make_async_copy(
                tab_hbm.at[idx_v.at[0, pl.ds(j * CHUNK, CHUNK)]],
                bufs[i].at[pl.ds(j * CHUNK, CHUNK)],
                gsem[i],
            ).wait()

    def scale_buf(i):
        def srow(r, _):
            for j in range(D // 16):
                sl = pl.ds(j * 16, 16)
                bufs[i][r, sl] = bufs[i][r, sl] * SCALE
            return ()

        lax.fori_loop(0, GROUP, srow, (), unroll=8)

    def fire_store(g, i):
        pltpu.async_copy(bufs[i], out_hbm.at[wid, g], ssem[i])

    def drain_store(i):
        pltpu.make_async_copy(bufs[i], out_hbm.at[wid, 0], ssem[i]).wait()

    for i in range(NBUF):
        fire_gather(i, i)

    def body(t, _):
        g0 = t * NBUF
        for i in range(NBUF):
            drain_gather(i)
            scale_buf(i)
            fire_store(g0 + i, i)
            if i >= 1:
                drain_store(i - 1)
                fire_gather(g0 + NBUF + (i - 1), i - 1)
        drain_store(NBUF - 1)
        fire_gather(g0 + NBUF + (NBUF - 1), NBUF - 1)
        return ()

    lax.fori_loop(0, NG // NBUF - 1, body, ())

    g0 = NG - NBUF
    for i in range(NBUF):
        drain_gather(i)
        scale_buf(i)
        fire_store(g0 + i, i)
    for i in range(NBUF):
        drain_store(i)


def kernel(x, table):
    xs = x.reshape(NW, NG, GROUP).astype(jnp.int32)
    out = _emb_lookup(xs, table)
    return out.reshape(B0, B1, D)


# speedup vs baseline: 1.1018x; 1.0012x over previous
import functools
import math

import jax
import jax.numpy as jnp
from jax import lax
from jax.experimental import pallas as pl
from jax.experimental.pallas import tpu as pltpu
from jax.experimental.pallas import tpu_sc as plsc

B0, B1 = 4096, 200
D = 64
NW = 32
ROWS_W = B0 // NW
NBUF = 4
SPLITS = ((0, 128), (128, 72))
SCALE = math.sqrt(D)

_mesh = plsc.VectorSubcoreMesh(core_axis_name="c", subcore_axis_name="s")


@functools.partial(
    pl.kernel,
    mesh=_mesh,
    compiler_params=pltpu.CompilerParams(use_tc_tiling_on_sc=False),
    out_type=jax.ShapeDtypeStruct((B0, B1, D), jnp.float32),
    scratch_types=[
        pltpu.VMEM((ROWS_W, B1), jnp.int32),
    ]
    + [pltpu.VMEM((B1, D), jnp.float32) for _ in range(NBUF)]
    + [pltpu.SemaphoreType.DMA for _ in range(2 * NBUF)],
)
def _emb_lookup(x_hbm, tab_hbm, out_hbm, idx_v, *bufs_and_sems):
    bufs = bufs_and_sems[:NBUF]
    gsem = bufs_and_sems[NBUF:2 * NBUF]
    ssem = bufs_and_sems[2 * NBUF:]
    wid = lax.axis_index("s") * 2 + lax.axis_index("c")
    row0 = wid * ROWS_W
    pltpu.sync_copy(x_hbm.at[pl.ds(row0, ROWS_W)], idx_v)

    def fire_gather(g, i):
        for off, n in SPLITS:
            pltpu.async_copy(
                tab_hbm.at[idx_v.at[g, pl.ds(off, n)]],
                bufs[i].at[pl.ds(off, n)],
                gsem[i],
            )

    def drain_gather(i):
        for off, n in SPLITS:
            pltpu.make_async_copy(
                tab_hbm.at[idx_v.at[0, pl.ds(off, n)]],
                bufs[i].at[pl.ds(off, n)],
                gsem[i],
            ).wait()

    def scale_buf(i):
        def srow(r, _):
            for j in range(D // 16):
                sl = pl.ds(j * 16, 16)
                bufs[i][r, sl] = bufs[i][r, sl] * SCALE
            return ()

        lax.fori_loop(0, B1, srow, (), unroll=8)

    def fire_store(g, i):
        pltpu.async_copy(bufs[i], out_hbm.at[row0 + g], ssem[i])

    def drain_store(i):
        pltpu.make_async_copy(bufs[i], out_hbm.at[0], ssem[i]).wait()

    for i in range(NBUF):
        fire_gather(i, i)

    def body(t, _):
        g0 = t * NBUF
        for i in range(NBUF):
            drain_gather(i)
            scale_buf(i)
            fire_store(g0 + i, i)
            if i >= 1:
                drain_store(i - 1)
                fire_gather(g0 + NBUF + (i - 1), i - 1)
        drain_store(NBUF - 1)
        fire_gather(g0 + NBUF + (NBUF - 1), NBUF - 1)
        return ()

    lax.fori_loop(0, ROWS_W // NBUF - 1, body, ())

    g0 = ROWS_W - NBUF
    for i in range(NBUF):
        drain_gather(i)
        scale_buf(i)
        fire_store(g0 + i, i)
    for i in range(NBUF):
        drain_store(i)


def kernel(x, table):
    return _emb_lookup(x.astype(jnp.int32), table)
